# Initial kernel scaffold; baseline (speedup 1.0000x reference)
import jax
import jax.numpy as jnp
from jax.experimental import pallas as pl


def kernel(edges, features, W0, b0, W1, b1, W2, b2):
    raise NotImplementedError("write your pallas kernel here")


# speedup vs baseline: 29.4783x; 29.4783x over previous
import functools

import jax
import jax.numpy as jnp
from jax import lax
from jax.experimental import pallas as pl
from jax.experimental.pallas import tpu as pltpu
from jax.experimental.pallas import tpu_sc as plsc

N = 10000
E = 320000
D_IN = 128
D_H = 128
D_OUT = 16

NC = 2
NS = 16
LW = 128
PER_TILE = E // (NC * NS)
CHUNKS = ((PER_TILE + LW - 1) // LW + 7) // 8 * 8
PT_PAD = CHUNKS * LW
PAD = PT_PAD - PER_TILE
NDUMMY = 240
NACC = N + NDUMMY
ACC_ROWS_PER_TILE = NACC // NS
DEGW = 16
SLAB = 640
TAIL_LO = 15 * SLAB
TAIL_ROWS = N - TAIL_LO


def _slab_copy(s, src_at, dst_at):
    @pl.when(s < NS - 1)
    def _():
        pltpu.sync_copy(*_slab_refs(src_at, dst_at, s * SLAB, SLAB))

    @pl.when(s == NS - 1)
    def _():
        pltpu.sync_copy(*_slab_refs(src_at, dst_at, TAIL_LO, TAIL_ROWS))


def _slab_refs(src_at, dst_at, lo, size):
    return src_at(lo, size), dst_at(lo, size)


def _sc_mesh():
    return plsc.VectorSubcoreMesh(core_axis_name="c", subcore_axis_name="s")


def _deg_body(dst_hbm, zeros_hbm, ones_hbm, out_hbm, dst_v, ones_v, acc_sh, sem):
    c = lax.axis_index("c")
    s = lax.axis_index("s")
    pltpu.sync_copy(dst_hbm.at[c, s], dst_v)
    pltpu.sync_copy(ones_hbm, ones_v)
    pltpu.sync_copy(
        zeros_hbm, acc_sh.at[pl.ds(s * ACC_ROWS_PER_TILE, ACC_ROWS_PER_TILE)]
    )
    plsc.subcore_barrier()
    def body(j, carry):
        pltpu.sync_copy(ones_v, acc_sh.at[dst_v.at[j]], add=True)
        return carry
    lax.fori_loop(0, CHUNKS, body, 0, unroll=4)
    plsc.subcore_barrier()
    pltpu.sync_copy(
        acc_sh.at[pl.ds(s * ACC_ROWS_PER_TILE, ACC_ROWS_PER_TILE)],
        out_hbm.at[c, pl.ds(s * ACC_ROWS_PER_TILE, ACC_ROWS_PER_TILE)],
    )


def _deg_call(dstp, zeros_deg, ones_deg):
    return pl.kernel(
        _deg_body,
        out_type=jax.ShapeDtypeStruct((NC, NACC, DEGW), jnp.float32),
        mesh=_sc_mesh(),
        compiler_params=pltpu.CompilerParams(use_tc_tiling_on_sc=False),
        scratch_types=[
            pltpu.VMEM((CHUNKS, LW), jnp.int32),
            pltpu.VMEM((LW, DEGW), jnp.float32),
            pltpu.VMEM_SHARED((NACC, DEGW), jnp.float32),
            pltpu.SemaphoreType.DMA,
        ],
    )(dstp, zeros_deg, ones_deg)


def _prop_body(p_hbm, src_hbm, dst_hbm, zeros_hbm, out_hbm,
               src_v, dst_v, rows0, rows1, acc_sh, sem0, sem1, *, stage):
    c = lax.axis_index("c")
    s = lax.axis_index("s")
    rows = (rows0, rows1)
    sems = (sem0, sem1)

    @pl.when(c == 0)
    def _():
        _slab_copy(s, lambda lo, sz: p_hbm.at[pl.ds(lo, sz)],
                   lambda lo, sz: acc_sh.at[pl.ds(lo, sz)])

    @pl.when(c == 1)
    def _():
        _slab_copy(s, lambda lo, sz: zeros_hbm.at[pl.ds(lo, sz)],
                   lambda lo, sz: acc_sh.at[pl.ds(lo, sz)])

    plsc.subcore_barrier()

    for h in range(CHUNKS // stage):
        pltpu.sync_copy(src_hbm.at[c, s, pl.ds(h * stage, stage)], src_v)
        pltpu.sync_copy(dst_hbm.at[c, s, pl.ds(h * stage, stage)], dst_v)
        pltpu.async_copy(p_hbm.at[src_v.at[0]], rows0, sem0)
        pltpu.async_copy(p_hbm.at[src_v.at[1]], rows1, sem1)

        def body(i, carry):
            for b in range(2):
                j = i * 2 + b
                pltpu.make_async_copy(
                    p_hbm.at[src_v.at[j]], rows[b], sems[b]
                ).wait()
                pltpu.sync_copy(rows[b], acc_sh.at[dst_v.at[j]], add=True)

                @pl.when(j + 2 < stage)
                def _():
                    pltpu.async_copy(p_hbm.at[src_v.at[j + 2]], rows[b], sems[b])
            return carry

        lax.fori_loop(0, stage // 2, body, 0)
    plsc.subcore_barrier()
    _slab_copy(s, lambda lo, sz: acc_sh.at[pl.ds(lo, sz)],
               lambda lo, sz: out_hbm.at[c, pl.ds(lo, sz)])


def _prop_call(p, srcp, dstp, zeros_nd):
    d = p.shape[1]
    stage = CHUNKS // 2 if d > 64 else CHUNKS
    params = None if d >= 128 else pltpu.CompilerParams(use_tc_tiling_on_sc=False)
    return pl.kernel(
        functools.partial(_prop_body, stage=stage),
        out_type=jax.ShapeDtypeStruct((NC, N, d), jnp.float32),
        mesh=_sc_mesh(),
        compiler_params=params,
        scratch_types=[
            pltpu.VMEM((stage, LW), jnp.int32),
            pltpu.VMEM((stage, LW), jnp.int32),
            pltpu.VMEM((LW, d), jnp.float32),
            pltpu.VMEM((LW, d), jnp.float32),
            pltpu.VMEM_SHARED((NACC, d), jnp.float32),
            pltpu.SemaphoreType.DMA,
            pltpu.SemaphoreType.DMA,
        ],
    )(p, srcp, dstp, zeros_nd)


_RB = 1000
_GRID = N // _RB


def _pre_body(feat_ref, w_ref, degp_ref, p_ref, dis_ref):
    dp = degp_ref[...]
    dis = lax.rsqrt(dp[0, :, 0:1] + dp[1, :, 0:1] + 1.0)
    h = jnp.dot(feat_ref[...], w_ref[...], preferred_element_type=jnp.float32)
    p_ref[...] = h * dis
    dis_ref[...] = dis


def _pre_call(features, w0, degp):
    return pl.pallas_call(
        _pre_body,
        grid=(_GRID,),
        in_specs=[
            pl.BlockSpec((_RB, D_IN), lambda i: (i, 0)),
            pl.BlockSpec((D_IN, D_H), lambda i: (0, 0)),
            pl.BlockSpec((NC, _RB, DEGW), lambda i: (0, i, 0)),
        ],
        out_specs=[
            pl.BlockSpec((_RB, D_H), lambda i: (i, 0)),
            pl.BlockSpec((_RB, 1), lambda i: (i, 0)),
        ],
        out_shape=[
            jax.ShapeDtypeStruct((N, D_H), jnp.float32),
            jax.ShapeDtypeStruct((N, 1), jnp.float32),
        ],
    )(features, w0, degp)


def _mid_body(sp_ref, dis_ref, b_ref, w_ref, p_ref):
    sp = sp_ref[...]
    dis = dis_ref[...]
    x = jax.nn.relu((sp[0] + sp[1]) * dis + b_ref[...][None, :])
    p_ref[...] = jnp.dot(x, w_ref[...], preferred_element_type=jnp.float32) * dis


def _mid_call(s_parts, dis, b, w):
    d_in, d_out = w.shape
    return pl.pallas_call(
        _mid_body,
        grid=(_GRID,),
        in_specs=[
            pl.BlockSpec((NC, _RB, d_in), lambda i: (0, i, 0)),
            pl.BlockSpec((_RB, 1), lambda i: (i, 0)),
            pl.BlockSpec((d_in,), lambda i: (0,)),
            pl.BlockSpec((d_in, d_out), lambda i: (0, 0)),
        ],
        out_specs=pl.BlockSpec((_RB, d_out), lambda i: (i, 0)),
        out_shape=jax.ShapeDtypeStruct((N, d_out), jnp.float32),
    )(s_parts, dis, b, w)


def _post_body(sp_ref, dis_ref, b_ref, out_ref):
    sp = sp_ref[...]
    z = (sp[0] + sp[1]) * dis_ref[...] + b_ref[...][None, :]
    m = jnp.max(z, axis=1, keepdims=True)
    lse = m + jnp.log(jnp.sum(jnp.exp(z - m), axis=1, keepdims=True))
    out_ref[...] = z - lse


def _post_call(s_parts, dis, b):
    return pl.pallas_call(
        _post_body,
        grid=(_GRID,),
        in_specs=[
            pl.BlockSpec((NC, _RB, D_OUT), lambda i: (0, i, 0)),
            pl.BlockSpec((_RB, 1), lambda i: (i, 0)),
            pl.BlockSpec((D_OUT,), lambda i: (0,)),
        ],
        out_specs=pl.BlockSpec((_RB, D_OUT), lambda i: (i, 0)),
        out_shape=jax.ShapeDtypeStruct((N, D_OUT), jnp.float32),
    )(s_parts, dis, b)


def kernel(edges, features, W0, b0, W1, b1, W2, b2):
    src = edges[0].astype(jnp.int32)
    dst = edges[1].astype(jnp.int32)

    pad_src = jnp.broadcast_to(
        (jnp.arange(PAD, dtype=jnp.int32) % 64)[None, :], (NC * NS, PAD)
    )
    pad_dst = jnp.broadcast_to(
        (N + jnp.arange(PAD, dtype=jnp.int32))[None, :], (NC * NS, PAD)
    )
    srcp = jnp.concatenate([src.reshape(NC * NS, PER_TILE), pad_src], axis=1)
    dstp = jnp.concatenate([dst.reshape(NC * NS, PER_TILE), pad_dst], axis=1)
    srcp = srcp.reshape(NC, NS, CHUNKS, LW)
    dstp = dstp.reshape(NC, NS, CHUNKS, LW)

    zeros_deg = jnp.zeros((ACC_ROWS_PER_TILE, DEGW), jnp.float32)
    ones_deg = jnp.ones((LW, DEGW), jnp.float32)
    zeros_h = jnp.zeros((N, D_H), jnp.float32)
    zeros_o = jnp.zeros((N, D_OUT), jnp.float32)

    degp = _deg_call(dstp, zeros_deg, ones_deg)
    degp = degp[:, :N, :]

    p0, dis = _pre_call(features, W0, degp)
    s0 = _prop_call(p0, srcp, dstp, zeros_h)
    p1 = _mid_call(s0, dis, b0, W1)
    s1 = _prop_call(p1, srcp, dstp, zeros_h)
    p2 = _mid_call(s1, dis, b1, W2)
    s2 = _prop_call(p2, srcp, dstp, zeros_o)
    return _post_call(s2, dis, b2)


# speedup vs baseline: 29.8139x; 1.0114x over previous
import functools

import jax
import jax.numpy as jnp
from jax import lax
from jax.experimental import pallas as pl
from jax.experimental.pallas import tpu as pltpu
from jax.experimental.pallas import tpu_sc as plsc

N = 10000
E = 320000
D_IN = 128
D_H = 128
D_OUT = 16

NC = 2
NS = 16
LW = 128
PER_TILE = E // (NC * NS)
CHUNKS = ((PER_TILE + LW - 1) // LW + 7) // 8 * 8
PT_PAD = CHUNKS * LW
PAD = PT_PAD - PER_TILE
NDUMMY = 240
NACC = N + NDUMMY
ACC_ROWS_PER_TILE = NACC // NS
DEGW = 16
SLAB = 640
TAIL_LO = 15 * SLAB
TAIL_ROWS = N - TAIL_LO


def _slab_copy(s, src_at, dst_at):
    @pl.when(s < NS - 1)
    def _():
        pltpu.sync_copy(*_slab_refs(src_at, dst_at, s * SLAB, SLAB))

    @pl.when(s == NS - 1)
    def _():
        pltpu.sync_copy(*_slab_refs(src_at, dst_at, TAIL_LO, TAIL_ROWS))


def _slab_refs(src_at, dst_at, lo, size):
    return src_at(lo, size), dst_at(lo, size)


def _sc_mesh():
    return plsc.VectorSubcoreMesh(core_axis_name="c", subcore_axis_name="s")


def _deg_body(dst_hbm, zeros_hbm, ones_hbm, out_hbm, dst_v, ones_v, acc_sh, sem):
    c = lax.axis_index("c")
    s = lax.axis_index("s")
    pltpu.sync_copy(dst_hbm.at[c, s], dst_v)
    pltpu.sync_copy(ones_hbm, ones_v)
    pltpu.sync_copy(
        zeros_hbm, acc_sh.at[pl.ds(s * ACC_ROWS_PER_TILE, ACC_ROWS_PER_TILE)]
    )
    plsc.subcore_barrier()
    def body(j, carry):
        pltpu.sync_copy(ones_v, acc_sh.at[dst_v.at[j]], add=True)
        return carry
    lax.fori_loop(0, CHUNKS, body, 0, unroll=4)
    plsc.subcore_barrier()
    pltpu.sync_copy(
        acc_sh.at[pl.ds(s * ACC_ROWS_PER_TILE, ACC_ROWS_PER_TILE)],
        out_hbm.at[c, pl.ds(s * ACC_ROWS_PER_TILE, ACC_ROWS_PER_TILE)],
    )


def _deg_call(dstp, zeros_deg, ones_deg):
    return pl.kernel(
        _deg_body,
        out_type=jax.ShapeDtypeStruct((NC, NACC, DEGW), jnp.float32),
        mesh=_sc_mesh(),
        compiler_params=pltpu.CompilerParams(use_tc_tiling_on_sc=False),
        scratch_types=[
            pltpu.VMEM((CHUNKS, LW), jnp.int32),
            pltpu.VMEM((LW, DEGW), jnp.float32),
            pltpu.VMEM_SHARED((NACC, DEGW), jnp.float32),
            pltpu.SemaphoreType.DMA,
        ],
    )(dstp, zeros_deg, ones_deg)


def _prop_body(p_hbm, src_hbm, dst_hbm, zeros_hbm, out_hbm,
               src_v, dst_v, rows0, rows1, acc_sh, sem0, sem1, *, stage):
    c = lax.axis_index("c")
    s = lax.axis_index("s")
    rows = (rows0, rows1)
    sems = (sem0, sem1)

    @pl.when(c == 0)
    def _():
        _slab_copy(s, lambda lo, sz: p_hbm.at[pl.ds(lo, sz)],
                   lambda lo, sz: acc_sh.at[pl.ds(lo, sz)])

    @pl.when(c == 1)
    def _():
        _slab_copy(s, lambda lo, sz: zeros_hbm.at[pl.ds(lo, sz)],
                   lambda lo, sz: acc_sh.at[pl.ds(lo, sz)])

    plsc.subcore_barrier()

    for h in range(CHUNKS // stage):
        pltpu.sync_copy(src_hbm.at[c, s, pl.ds(h * stage, stage)], src_v)
        pltpu.sync_copy(dst_hbm.at[c, s, pl.ds(h * stage, stage)], dst_v)
        pltpu.async_copy(p_hbm.at[src_v.at[0]], rows0, sem0)
        pltpu.async_copy(p_hbm.at[src_v.at[1]], rows1, sem1)

        def body(i, carry):
            for b in range(2):
                j = i * 2 + b
                pltpu.make_async_copy(
                    p_hbm.at[src_v.at[j]], rows[b], sems[b]
                ).wait()
                pltpu.sync_copy(rows[b], acc_sh.at[dst_v.at[j]], add=True)

                @pl.when(j + 2 < stage)
                def _():
                    pltpu.async_copy(p_hbm.at[src_v.at[j + 2]], rows[b], sems[b])
            return carry

        lax.fori_loop(0, stage // 2, body, 0)
    plsc.subcore_barrier()
    _slab_copy(s, lambda lo, sz: acc_sh.at[pl.ds(lo, sz)],
               lambda lo, sz: out_hbm.at[c, pl.ds(lo, sz)])


def _prop_call(p, srcp, dstp, zeros_nd):
    d = p.shape[1]
    stage = CHUNKS // 2 if d > 64 else CHUNKS
    params = None if d >= 128 else pltpu.CompilerParams(use_tc_tiling_on_sc=False)
    return pl.kernel(
        functools.partial(_prop_body, stage=stage),
        out_type=jax.ShapeDtypeStruct((NC, N, d), jnp.float32),
        mesh=_sc_mesh(),
        compiler_params=params,
        scratch_types=[
            pltpu.VMEM((stage, LW), jnp.int32),
            pltpu.VMEM((stage, LW), jnp.int32),
            pltpu.VMEM((LW, d), jnp.float32),
            pltpu.VMEM((LW, d), jnp.float32),
            pltpu.VMEM_SHARED((NACC, d), jnp.float32),
            pltpu.SemaphoreType.DMA,
            pltpu.SemaphoreType.DMA,
        ],
    )(p, srcp, dstp, zeros_nd)


_RB = 1000
_GRID = N // _RB


def _mm0_body(feat_ref, w_ref, h_ref):
    h_ref[...] = jnp.dot(
        feat_ref[...], w_ref[...], preferred_element_type=jnp.float32
    )


def _mm0_call(features, w0):
    return pl.pallas_call(
        _mm0_body,
        grid=(_GRID,),
        in_specs=[
            pl.BlockSpec((_RB, D_IN), lambda i: (i, 0)),
            pl.BlockSpec((D_IN, D_H), lambda i: (0, 0)),
        ],
        out_specs=pl.BlockSpec((_RB, D_H), lambda i: (i, 0)),
        out_shape=jax.ShapeDtypeStruct((N, D_H), jnp.float32),
    )(features, w0)


def _pre_body(h_ref, degp_ref, p_ref, dis_ref):
    dp = degp_ref[...]
    dis = lax.rsqrt(dp[0, :, 0:1] + dp[1, :, 0:1] + 1.0)
    p_ref[...] = h_ref[...] * dis
    dis_ref[...] = dis


def _pre_call(h0, degp):
    return pl.pallas_call(
        _pre_body,
        grid=(_GRID,),
        in_specs=[
            pl.BlockSpec((_RB, D_H), lambda i: (i, 0)),
            pl.BlockSpec((NC, _RB, DEGW), lambda i: (0, i, 0)),
        ],
        out_specs=[
            pl.BlockSpec((_RB, D_H), lambda i: (i, 0)),
            pl.BlockSpec((_RB, 1), lambda i: (i, 0)),
        ],
        out_shape=[
            jax.ShapeDtypeStruct((N, D_H), jnp.float32),
            jax.ShapeDtypeStruct((N, 1), jnp.float32),
        ],
    )(h0, degp)


def _mid_body(sp_ref, dis_ref, b_ref, w_ref, p_ref):
    sp = sp_ref[...]
    dis = dis_ref[...]
    x = jax.nn.relu((sp[0] + sp[1]) * dis + b_ref[...][None, :])
    p_ref[...] = jnp.dot(x, w_ref[...], preferred_element_type=jnp.float32) * dis


def _mid_call(s_parts, dis, b, w):
    d_in, d_out = w.shape
    return pl.pallas_call(
        _mid_body,
        grid=(_GRID,),
        in_specs=[
            pl.BlockSpec((NC, _RB, d_in), lambda i: (0, i, 0)),
            pl.BlockSpec((_RB, 1), lambda i: (i, 0)),
            pl.BlockSpec((d_in,), lambda i: (0,)),
            pl.BlockSpec((d_in, d_out), lambda i: (0, 0)),
        ],
        out_specs=pl.BlockSpec((_RB, d_out), lambda i: (i, 0)),
        out_shape=jax.ShapeDtypeStruct((N, d_out), jnp.float32),
    )(s_parts, dis, b, w)


def _post_body(sp_ref, dis_ref, b_ref, out_ref):
    sp = sp_ref[...]
    z = (sp[0] + sp[1]) * dis_ref[...] + b_ref[...][None, :]
    m = jnp.max(z, axis=1, keepdims=True)
    lse = m + jnp.log(jnp.sum(jnp.exp(z - m), axis=1, keepdims=True))
    out_ref[...] = z - lse


def _post_call(s_parts, dis, b):
    return pl.pallas_call(
        _post_body,
        grid=(_GRID,),
        in_specs=[
            pl.BlockSpec((NC, _RB, D_OUT), lambda i: (0, i, 0)),
            pl.BlockSpec((_RB, 1), lambda i: (i, 0)),
            pl.BlockSpec((D_OUT,), lambda i: (0,)),
        ],
        out_specs=pl.BlockSpec((_RB, D_OUT), lambda i: (i, 0)),
        out_shape=jax.ShapeDtypeStruct((N, D_OUT), jnp.float32),
    )(s_parts, dis, b)


def kernel(edges, features, W0, b0, W1, b1, W2, b2):
    src = edges[0].astype(jnp.int32)
    dst = edges[1].astype(jnp.int32)

    pad_src = jnp.broadcast_to(
        (jnp.arange(PAD, dtype=jnp.int32) % 64)[None, :], (NC * NS, PAD)
    )
    pad_dst = jnp.broadcast_to(
        (N + jnp.arange(PAD, dtype=jnp.int32))[None, :], (NC * NS, PAD)
    )
    srcp = jnp.concatenate([src.reshape(NC * NS, PER_TILE), pad_src], axis=1)
    dstp = jnp.concatenate([dst.reshape(NC * NS, PER_TILE), pad_dst], axis=1)
    srcp = srcp.reshape(NC, NS, CHUNKS, LW)
    dstp = dstp.reshape(NC, NS, CHUNKS, LW)

    zeros_deg = jnp.zeros((ACC_ROWS_PER_TILE, DEGW), jnp.float32)
    ones_deg = jnp.ones((LW, DEGW), jnp.float32)
    zeros_h = jnp.zeros((N, D_H), jnp.float32)
    zeros_o = jnp.zeros((N, D_OUT), jnp.float32)

    degp = _deg_call(dstp, zeros_deg, ones_deg)
    h0 = _mm0_call(features, W0)
    p0, dis = _pre_call(h0, degp)
    s0 = _prop_call(p0, srcp, dstp, zeros_h)
    p1 = _mid_call(s0, dis, b0, W1)
    s1 = _prop_call(p1, srcp, dstp, zeros_h)
    p2 = _mid_call(s1, dis, b1, W2)
    s2 = _prop_call(p2, srcp, dstp, zeros_o)
    return _post_call(s2, dis, b2)


# speedup vs baseline: 30.1869x; 1.0125x over previous
import functools

import jax
import jax.numpy as jnp
from jax import lax
from jax.experimental import pallas as pl
from jax.experimental.pallas import tpu as pltpu
from jax.experimental.pallas import tpu_sc as plsc

N = 10000
E = 320000
D_IN = 128
D_H = 128
D_OUT = 16

NC = 2
NS = 16
LW = 128
PER_TILE = E // (NC * NS)
CHUNKS = ((PER_TILE + LW - 1) // LW + 7) // 8 * 8
PT_PAD = CHUNKS * LW
PAD = PT_PAD - PER_TILE
NDUMMY = 240
NACC = N + NDUMMY
ACC_ROWS_PER_TILE = NACC // NS
DEGW = 16
SLAB = 640
TAIL_LO = 15 * SLAB
TAIL_ROWS = N - TAIL_LO


def _slab_copy(s, src_at, dst_at):
    @pl.when(s < NS - 1)
    def _():
        pltpu.sync_copy(*_slab_refs(src_at, dst_at, s * SLAB, SLAB))

    @pl.when(s == NS - 1)
    def _():
        pltpu.sync_copy(*_slab_refs(src_at, dst_at, TAIL_LO, TAIL_ROWS))


def _slab_refs(src_at, dst_at, lo, size):
    return src_at(lo, size), dst_at(lo, size)


def _sc_mesh():
    return plsc.VectorSubcoreMesh(core_axis_name="c", subcore_axis_name="s")


def _deg_body(dst_hbm, zeros_hbm, ones_hbm, out_hbm, dst_v, ones_v, acc_sh, sem):
    c = lax.axis_index("c")
    s = lax.axis_index("s")
    pltpu.sync_copy(dst_hbm.at[c, s], dst_v)
    pltpu.sync_copy(ones_hbm, ones_v)
    pltpu.sync_copy(
        zeros_hbm, acc_sh.at[pl.ds(s * ACC_ROWS_PER_TILE, ACC_ROWS_PER_TILE)]
    )
    plsc.subcore_barrier()
    def body(j, carry):
        pltpu.sync_copy(ones_v, acc_sh.at[dst_v.at[j]], add=True)
        return carry
    lax.fori_loop(0, CHUNKS, body, 0, unroll=4)
    plsc.subcore_barrier()
    pltpu.sync_copy(
        acc_sh.at[pl.ds(s * ACC_ROWS_PER_TILE, ACC_ROWS_PER_TILE)],
        out_hbm.at[c, pl.ds(s * ACC_ROWS_PER_TILE, ACC_ROWS_PER_TILE)],
    )


def _deg_call(dstp, zeros_deg, ones_deg):
    return pl.kernel(
        _deg_body,
        out_type=jax.ShapeDtypeStruct((NC, NACC, DEGW), jnp.float32),
        mesh=_sc_mesh(),
        compiler_params=pltpu.CompilerParams(use_tc_tiling_on_sc=False),
        scratch_types=[
            pltpu.VMEM((CHUNKS, LW), jnp.int32),
            pltpu.VMEM((LW, DEGW), jnp.float32),
            pltpu.VMEM_SHARED((NACC, DEGW), jnp.float32),
            pltpu.SemaphoreType.DMA,
        ],
    )(dstp, zeros_deg, ones_deg)


def _prop_body(p_hbm, src_hbm, dst_hbm, zeros_hbm, out_hbm,
               src_v, dst_v, rows0, rows1, acc_sh, sem0, sem1, *, stage):
    c = lax.axis_index("c")
    s = lax.axis_index("s")
    rows = (rows0, rows1)
    sems = (sem0, sem1)

    @pl.when(c == 0)
    def _():
        _slab_copy(s, lambda lo, sz: p_hbm.at[pl.ds(lo, sz)],
                   lambda lo, sz: acc_sh.at[pl.ds(lo, sz)])

    @pl.when(c == 1)
    def _():
        _slab_copy(s, lambda lo, sz: zeros_hbm.at[pl.ds(lo, sz)],
                   lambda lo, sz: acc_sh.at[pl.ds(lo, sz)])

    plsc.subcore_barrier()

    for h in range(CHUNKS // stage):
        pltpu.sync_copy(src_hbm.at[c, s, pl.ds(h * stage, stage)], src_v)
        pltpu.sync_copy(dst_hbm.at[c, s, pl.ds(h * stage, stage)], dst_v)
        pltpu.async_copy(p_hbm.at[src_v.at[0]], rows0, sem0)
        pltpu.async_copy(p_hbm.at[src_v.at[1]], rows1, sem1)

        def body(i, carry):
            for b in range(2):
                j = i * 2 + b
                pltpu.make_async_copy(
                    p_hbm.at[src_v.at[j]], rows[b], sems[b]
                ).wait()
                pltpu.sync_copy(rows[b], acc_sh.at[dst_v.at[j]], add=True)

                @pl.when(j + 2 < stage)
                def _():
                    pltpu.async_copy(p_hbm.at[src_v.at[j + 2]], rows[b], sems[b])
            return carry

        lax.fori_loop(0, stage // 2, body, 0)
    plsc.subcore_barrier()
    _slab_copy(s, lambda lo, sz: acc_sh.at[pl.ds(lo, sz)],
               lambda lo, sz: out_hbm.at[c, pl.ds(lo, sz)])


def _prop_call(p, srcp, dstp, zeros_nd):
    d = p.shape[1]
    stage = CHUNKS // 2 if d > 64 else CHUNKS
    params = None if d >= 128 else pltpu.CompilerParams(use_tc_tiling_on_sc=False)
    return pl.kernel(
        functools.partial(_prop_body, stage=stage),
        out_type=jax.ShapeDtypeStruct((NC, N, d), jnp.float32),
        mesh=_sc_mesh(),
        compiler_params=params,
        scratch_types=[
            pltpu.VMEM((stage, LW), jnp.int32),
            pltpu.VMEM((stage, LW), jnp.int32),
            pltpu.VMEM((LW, d), jnp.float32),
            pltpu.VMEM((LW, d), jnp.float32),
            pltpu.VMEM_SHARED((NACC, d), jnp.float32),
            pltpu.SemaphoreType.DMA,
            pltpu.SemaphoreType.DMA,
        ],
    )(p, srcp, dstp, zeros_nd)


_RB = 2000
_GRID = N // _RB


def _mm0_body(feat_ref, w_ref, h_ref):
    h_ref[...] = jnp.dot(
        feat_ref[...], w_ref[...], preferred_element_type=jnp.float32
    )


def _mm0_call(features, w0):
    return pl.pallas_call(
        _mm0_body,
        grid=(_GRID,),
        in_specs=[
            pl.BlockSpec((_RB, D_IN), lambda i: (i, 0)),
            pl.BlockSpec((D_IN, D_H), lambda i: (0, 0)),
        ],
        out_specs=pl.BlockSpec((_RB, D_H), lambda i: (i, 0)),
        out_shape=jax.ShapeDtypeStruct((N, D_H), jnp.float32),
    )(features, w0)


def _pre_body(h_ref, degp_ref, p_ref, dis_ref):
    dp = degp_ref[...]
    dis = lax.rsqrt(dp[0, :, 0:1] + dp[1, :, 0:1] + 1.0)
    p_ref[...] = h_ref[...] * dis
    dis_ref[...] = dis


def _pre_call(h0, degp):
    return pl.pallas_call(
        _pre_body,
        grid=(_GRID,),
        in_specs=[
            pl.BlockSpec((_RB, D_H), lambda i: (i, 0)),
            pl.BlockSpec((NC, _RB, DEGW), lambda i: (0, i, 0)),
        ],
        out_specs=[
            pl.BlockSpec((_RB, D_H), lambda i: (i, 0)),
            pl.BlockSpec((_RB, 1), lambda i: (i, 0)),
        ],
        out_shape=[
            jax.ShapeDtypeStruct((N, D_H), jnp.float32),
            jax.ShapeDtypeStruct((N, 1), jnp.float32),
        ],
    )(h0, degp)


def _mid_body(sp_ref, dis_ref, b_ref, w_ref, p_ref):
    sp = sp_ref[...]
    dis = dis_ref[...]
    x = jax.nn.relu((sp[0] + sp[1]) * dis + b_ref[...][None, :])
    p_ref[...] = jnp.dot(x, w_ref[...], preferred_element_type=jnp.float32) * dis


def _mid_call(s_parts, dis, b, w):
    d_in, d_out = w.shape
    return pl.pallas_call(
        _mid_body,
        grid=(_GRID,),
        in_specs=[
            pl.BlockSpec((NC, _RB, d_in), lambda i: (0, i, 0)),
            pl.BlockSpec((_RB, 1), lambda i: (i, 0)),
            pl.BlockSpec((d_in,), lambda i: (0,)),
            pl.BlockSpec((d_in, d_out), lambda i: (0, 0)),
        ],
        out_specs=pl.BlockSpec((_RB, d_out), lambda i: (i, 0)),
        out_shape=jax.ShapeDtypeStruct((N, d_out), jnp.float32),
    )(s_parts, dis, b, w)


def _post_body(sp_ref, dis_ref, b_ref, out_ref):
    sp = sp_ref[...]
    z = (sp[0] + sp[1]) * dis_ref[...] + b_ref[...][None, :]
    m = jnp.max(z, axis=1, keepdims=True)
    lse = m + jnp.log(jnp.sum(jnp.exp(z - m), axis=1, keepdims=True))
    out_ref[...] = z - lse


def _post_call(s_parts, dis, b):
    return pl.pallas_call(
        _post_body,
        grid=(_GRID,),
        in_specs=[
            pl.BlockSpec((NC, _RB, D_OUT), lambda i: (0, i, 0)),
            pl.BlockSpec((_RB, 1), lambda i: (i, 0)),
            pl.BlockSpec((D_OUT,), lambda i: (0,)),
        ],
        out_specs=pl.BlockSpec((_RB, D_OUT), lambda i: (i, 0)),
        out_shape=jax.ShapeDtypeStruct((N, D_OUT), jnp.float32),
    )(s_parts, dis, b)


def kernel(edges, features, W0, b0, W1, b1, W2, b2):
    src = edges[0].astype(jnp.int32)
    dst = edges[1].astype(jnp.int32)

    pad_src = jnp.broadcast_to(
        (jnp.arange(PAD, dtype=jnp.int32) % 64)[None, :], (NC * NS, PAD)
    )
    pad_dst = jnp.broadcast_to(
        (N + jnp.arange(PAD, dtype=jnp.int32))[None, :], (NC * NS, PAD)
    )
    srcp = (
        jnp.full((NC * NS, PT_PAD), 0, jnp.int32)
        .at[:, PER_TILE:].set(pad_src)
        .at[:, :PER_TILE].set(src.reshape(NC * NS, PER_TILE))
        .reshape(NC, NS, CHUNKS, LW)
    )
    dstp = (
        jnp.full((NC * NS, PT_PAD), 0, jnp.int32)
        .at[:, PER_TILE:].set(pad_dst)
        .at[:, :PER_TILE].set(dst.reshape(NC * NS, PER_TILE))
        .reshape(NC, NS, CHUNKS, LW)
    )

    zeros_deg = jnp.zeros((ACC_ROWS_PER_TILE, DEGW), jnp.float32)
    ones_deg = jnp.ones((LW, DEGW), jnp.float32)
    zeros_h = jnp.zeros((N, D_H), jnp.float32)
    zeros_o = jnp.zeros((N, D_OUT), jnp.float32)

    degp = _deg_call(dstp, zeros_deg, ones_deg)
    h0 = _mm0_call(features, W0)
    p0, dis = _pre_call(h0, degp)
    s0 = _prop_call(p0, srcp, dstp, zeros_h)
    p1 = _mid_call(s0, dis, b0, W1)
    s1 = _prop_call(p1, srcp, dstp, zeros_h)
    p2 = _mid_call(s1, dis, b1, W2)
    s2 = _prop_call(p2, srcp, dstp, zeros_o)
    return _post_call(s2, dis, b2)


# speedup vs baseline: 32.3910x; 1.0730x over previous
import functools

import jax
import jax.numpy as jnp
from jax import lax
from jax.experimental import pallas as pl
from jax.experimental.pallas import tpu as pltpu
from jax.experimental.pallas import tpu_sc as plsc

N = 10000
E = 320000
D_IN = 128
D_H = 128
D_OUT = 16

NC = 2
NS = 16
LW = 128
PER_TILE = E // (NC * NS)
CHUNKS = ((PER_TILE + LW - 1) // LW + 7) // 8 * 8
PT_PAD = CHUNKS * LW
PAD = PT_PAD - PER_TILE
NDUMMY = 240
NACC = N + NDUMMY
ACC_ROWS_PER_TILE = NACC // NS
DEGW = 16
SLAB = 640
TAIL_LO = 15 * SLAB
TAIL_ROWS = N - TAIL_LO


def _slab_copy(s, src_at, dst_at):
    @pl.when(s < NS - 1)
    def _():
        pltpu.sync_copy(*_slab_refs(src_at, dst_at, s * SLAB, SLAB))

    @pl.when(s == NS - 1)
    def _():
        pltpu.sync_copy(*_slab_refs(src_at, dst_at, TAIL_LO, TAIL_ROWS))


def _slab_refs(src_at, dst_at, lo, size):
    return src_at(lo, size), dst_at(lo, size)


def _sc_mesh():
    return plsc.VectorSubcoreMesh(core_axis_name="c", subcore_axis_name="s")


def _deg_body(dst_hbm, zeros_hbm, ones_hbm, out_hbm, dst_v, ones_v, acc_sh, sem):
    c = lax.axis_index("c")
    s = lax.axis_index("s")
    pltpu.sync_copy(dst_hbm.at[c, s], dst_v)
    pltpu.sync_copy(ones_hbm, ones_v)
    pltpu.sync_copy(
        zeros_hbm, acc_sh.at[pl.ds(s * ACC_ROWS_PER_TILE, ACC_ROWS_PER_TILE)]
    )
    plsc.subcore_barrier()
    def body(j, carry):
        pltpu.sync_copy(ones_v, acc_sh.at[dst_v.at[j]], add=True)
        return carry
    lax.fori_loop(0, CHUNKS, body, 0, unroll=4)
    plsc.subcore_barrier()
    pltpu.sync_copy(
        acc_sh.at[pl.ds(s * ACC_ROWS_PER_TILE, ACC_ROWS_PER_TILE)],
        out_hbm.at[c, pl.ds(s * ACC_ROWS_PER_TILE, ACC_ROWS_PER_TILE)],
    )


def _deg_call(dstp, zeros_deg, ones_deg):
    return pl.kernel(
        _deg_body,
        out_type=jax.ShapeDtypeStruct((NC, NACC, DEGW), jnp.float32),
        mesh=_sc_mesh(),
        compiler_params=pltpu.CompilerParams(use_tc_tiling_on_sc=False),
        scratch_types=[
            pltpu.VMEM((CHUNKS, LW), jnp.int32),
            pltpu.VMEM((LW, DEGW), jnp.float32),
            pltpu.VMEM_SHARED((NACC, DEGW), jnp.float32),
            pltpu.SemaphoreType.DMA,
        ],
    )(dstp, zeros_deg, ones_deg)


def _prop_body(p_hbm, src_hbm, dst_hbm, zeros_hbm, out_hbm,
               src_v, dst_v, rows0, rows1, acc_sh, sem0, sem1, *, stage):
    c = lax.axis_index("c")
    s = lax.axis_index("s")
    rows = (rows0, rows1)
    sems = (sem0, sem1)

    @pl.when(c == 0)
    def _():
        _slab_copy(s, lambda lo, sz: p_hbm.at[pl.ds(lo, sz)],
                   lambda lo, sz: acc_sh.at[pl.ds(lo, sz)])

    @pl.when(c == 1)
    def _():
        _slab_copy(s, lambda lo, sz: zeros_hbm.at[pl.ds(lo, sz)],
                   lambda lo, sz: acc_sh.at[pl.ds(lo, sz)])

    plsc.subcore_barrier()

    for h in range(CHUNKS // stage):
        pltpu.sync_copy(src_hbm.at[c, s, pl.ds(h * stage, stage)], src_v)
        pltpu.sync_copy(dst_hbm.at[c, s, pl.ds(h * stage, stage)], dst_v)
        pltpu.async_copy(p_hbm.at[src_v.at[0]], rows0, sem0)
        pltpu.async_copy(p_hbm.at[src_v.at[1]], rows1, sem1)

        def body(i, carry):
            for b in range(2):
                j = i * 2 + b
                pltpu.make_async_copy(
                    p_hbm.at[src_v.at[j]], rows[b], sems[b]
                ).wait()
                pltpu.sync_copy(rows[b], acc_sh.at[dst_v.at[j]], add=True)

                @pl.when(j + 2 < stage)
                def _():
                    pltpu.async_copy(p_hbm.at[src_v.at[j + 2]], rows[b], sems[b])
            return carry

        lax.fori_loop(0, stage // 2, body, 0)
    plsc.subcore_barrier()
    _slab_copy(s, lambda lo, sz: acc_sh.at[pl.ds(lo, sz)],
               lambda lo, sz: out_hbm.at[c, pl.ds(lo, sz)])


def _prop_call(p, srcp, dstp, zeros_nd):
    d = p.shape[1]
    dt = p.dtype
    stage = CHUNKS // 2 if (d > 64 and dt == jnp.float32) else CHUNKS
    native = d >= 128 and dt == jnp.float32
    params = None if native else pltpu.CompilerParams(use_tc_tiling_on_sc=False)
    return pl.kernel(
        functools.partial(_prop_body, stage=stage),
        out_type=jax.ShapeDtypeStruct((NC, N, d), dt),
        mesh=_sc_mesh(),
        compiler_params=params,
        scratch_types=[
            pltpu.VMEM((stage, LW), jnp.int32),
            pltpu.VMEM((stage, LW), jnp.int32),
            pltpu.VMEM((LW, d), dt),
            pltpu.VMEM((LW, d), dt),
            pltpu.VMEM_SHARED((NACC, d), dt),
            pltpu.SemaphoreType.DMA,
            pltpu.SemaphoreType.DMA,
        ],
    )(p, srcp, dstp, zeros_nd)


_RB = 2000
_GRID = N // _RB


def _mm0_body(feat_ref, w_ref, h_ref):
    h_ref[...] = jnp.dot(
        feat_ref[...], w_ref[...], preferred_element_type=jnp.float32
    )


def _mm0_call(features, w0):
    return pl.pallas_call(
        _mm0_body,
        grid=(_GRID,),
        in_specs=[
            pl.BlockSpec((_RB, D_IN), lambda i: (i, 0)),
            pl.BlockSpec((D_IN, D_H), lambda i: (0, 0)),
        ],
        out_specs=pl.BlockSpec((_RB, D_H), lambda i: (i, 0)),
        out_shape=jax.ShapeDtypeStruct((N, D_H), jnp.float32),
    )(features, w0)


def _pre_body(h_ref, degp_ref, p_ref, dis_ref):
    dp = degp_ref[...]
    dis = lax.rsqrt(dp[0, :, 0:1] + dp[1, :, 0:1] + 1.0)
    p_ref[...] = (h_ref[...] * dis).astype(p_ref.dtype)
    dis_ref[...] = dis


def _pre_call(h0, degp):
    return pl.pallas_call(
        _pre_body,
        grid=(_GRID,),
        in_specs=[
            pl.BlockSpec((_RB, D_H), lambda i: (i, 0)),
            pl.BlockSpec((NC, _RB, DEGW), lambda i: (0, i, 0)),
        ],
        out_specs=[
            pl.BlockSpec((_RB, D_H), lambda i: (i, 0)),
            pl.BlockSpec((_RB, 1), lambda i: (i, 0)),
        ],
        out_shape=[
            jax.ShapeDtypeStruct((N, D_H), jnp.bfloat16),
            jax.ShapeDtypeStruct((N, 1), jnp.float32),
        ],
    )(h0, degp)


def _mid_body(sp_ref, dis_ref, b_ref, w_ref, p_ref):
    sp = sp_ref[...].astype(jnp.float32)
    dis = dis_ref[...]
    x = jax.nn.relu((sp[0] + sp[1]) * dis + b_ref[...][None, :])
    p = jnp.dot(x, w_ref[...], preferred_element_type=jnp.float32) * dis
    p_ref[...] = p.astype(p_ref.dtype)


def _mid_call(s_parts, dis, b, w, out_dtype):
    d_in, d_out = w.shape
    return pl.pallas_call(
        _mid_body,
        grid=(_GRID,),
        in_specs=[
            pl.BlockSpec((NC, _RB, d_in), lambda i: (0, i, 0)),
            pl.BlockSpec((_RB, 1), lambda i: (i, 0)),
            pl.BlockSpec((d_in,), lambda i: (0,)),
            pl.BlockSpec((d_in, d_out), lambda i: (0, 0)),
        ],
        out_specs=pl.BlockSpec((_RB, d_out), lambda i: (i, 0)),
        out_shape=jax.ShapeDtypeStruct((N, d_out), out_dtype),
    )(s_parts, dis, b, w)


def _post_body(sp_ref, dis_ref, b_ref, out_ref):
    sp = sp_ref[...].astype(jnp.float32)
    z = (sp[0] + sp[1]) * dis_ref[...] + b_ref[...][None, :]
    m = jnp.max(z, axis=1, keepdims=True)
    lse = m + jnp.log(jnp.sum(jnp.exp(z - m), axis=1, keepdims=True))
    out_ref[...] = z - lse


def _post_call(s_parts, dis, b):
    return pl.pallas_call(
        _post_body,
        grid=(_GRID,),
        in_specs=[
            pl.BlockSpec((NC, _RB, D_OUT), lambda i: (0, i, 0)),
            pl.BlockSpec((_RB, 1), lambda i: (i, 0)),
            pl.BlockSpec((D_OUT,), lambda i: (0,)),
        ],
        out_specs=pl.BlockSpec((_RB, D_OUT), lambda i: (i, 0)),
        out_shape=jax.ShapeDtypeStruct((N, D_OUT), jnp.float32),
    )(s_parts, dis, b)


def kernel(edges, features, W0, b0, W1, b1, W2, b2):
    src = edges[0].astype(jnp.int32)
    dst = edges[1].astype(jnp.int32)

    pad_src = jnp.broadcast_to(
        (jnp.arange(PAD, dtype=jnp.int32) % 64)[None, :], (NC * NS, PAD)
    )
    pad_dst = jnp.broadcast_to(
        (N + jnp.arange(PAD, dtype=jnp.int32))[None, :], (NC * NS, PAD)
    )
    srcp = (
        jnp.full((NC * NS, PT_PAD), 0, jnp.int32)
        .at[:, PER_TILE:].set(pad_src)
        .at[:, :PER_TILE].set(src.reshape(NC * NS, PER_TILE))
        .reshape(NC, NS, CHUNKS, LW)
    )
    dstp = (
        jnp.full((NC * NS, PT_PAD), 0, jnp.int32)
        .at[:, PER_TILE:].set(pad_dst)
        .at[:, :PER_TILE].set(dst.reshape(NC * NS, PER_TILE))
        .reshape(NC, NS, CHUNKS, LW)
    )

    zeros_deg = jnp.zeros((ACC_ROWS_PER_TILE, DEGW), jnp.float32)
    ones_deg = jnp.ones((LW, DEGW), jnp.float32)
    zeros_h = jnp.zeros((N, D_H), jnp.bfloat16)
    zeros_o = jnp.zeros((N, D_OUT), jnp.float32)

    degp = _deg_call(dstp, zeros_deg, ones_deg)
    h0 = _mm0_call(features, W0)
    p0, dis = _pre_call(h0, degp)
    s0 = _prop_call(p0, srcp, dstp, zeros_h)
    p1 = _mid_call(s0, dis, b0, W1, jnp.bfloat16)
    s1 = _prop_call(p1, srcp, dstp, zeros_h)
    p2 = _mid_call(s1, dis, b1, W2, jnp.float32)
    s2 = _prop_call(p2, srcp, dstp, zeros_o)
    return _post_call(s2, dis, b2)
